# baseline (speedup 1.0000x reference)
import functools

import jax
import jax.numpy as jnp
from jax import lax
from jax.experimental import pallas as pl
from jax.experimental.pallas import tpu as pltpu
from jax.experimental.pallas import tpu_sc as plsc

N = 65536
D = 256
OFF = 6

_NC = 2
_NS = 16
_NW = _NC * _NS
_L = 16
_SLAB = N // _NW


def _mask_body(idx_hbm, mask_hbm, idx_v, slab_v):
    wid = lax.axis_index("s") * _NC + lax.axis_index("c")
    lo = wid * _SLAB

    pltpu.sync_copy(idx_hbm, idx_v)

    unroll = 8
    zeros = jnp.zeros((_L,), jnp.float32)

    def zero_body(i, carry):
        for u in range(unroll):
            slab_v[pl.ds((i * unroll + u) * _L, _L)] = zeros
        return carry

    lax.fori_loop(0, _SLAB // (_L * unroll), zero_body, 0)

    ones = jnp.ones((_L,), jnp.float32)
    n_idx = idx_v.shape[0]

    def scatter_body(i, carry):
        for u in range(unroll):
            v = idx_v[pl.ds((i * unroll + u) * _L, _L)]
            rel = v - lo
            m = plsc.bitcast(rel, jnp.uint32) < _SLAB
            plsc.store_scatter(slab_v, [rel & (_SLAB - 1)], ones, mask=m)
        return carry

    lax.fori_loop(0, n_idx // (_L * unroll), scatter_body, 0)

    pltpu.sync_copy(slab_v, mask_hbm.at[pl.ds(lo, _SLAB)])


def _build_mask(indices):
    n_idx = indices.shape[0]
    mesh = plsc.VectorSubcoreMesh(core_axis_name="c", subcore_axis_name="s")
    kern = pl.kernel(
        _mask_body,
        out_type=jax.ShapeDtypeStruct((N,), jnp.float32),
        mesh=mesh,
        scratch_types=[
            pltpu.VMEM((n_idx,), jnp.int32),
            pltpu.VMEM((_SLAB,), jnp.float32),
        ],
        compiler_params=pltpu.CompilerParams(needs_layout_passes=False),
    )
    return kern(indices)


def _flip_body(x_ref, m_ref, o_ref):
    H = D // 2
    xb = x_ref[...]
    a = xb[:, :H]
    b = xb[:, H:]
    k = lax.broadcasted_iota(jnp.int32, a.shape, 1)
    idxg = jnp.where(k < OFF, OFF - 1 - k, H + OFF - 1 - k)
    ga = jnp.take_along_axis(a, idxg, axis=1)
    gb = jnp.take_along_axis(b, idxg, axis=1)
    out_a = jnp.where(k < OFF, a, gb)
    out_b = jnp.where(k < OFF, gb, ga)
    shifted = jnp.concatenate([out_a, out_b], axis=1)
    rows = xb.shape[0]
    m2d = m_ref[...]
    mrep = jnp.broadcast_to(m2d[:, None, :], (rows // 128, 128, 128))
    mrep = mrep.reshape(rows, 128)
    rlane = lax.broadcasted_iota(jnp.int32, (rows, 128), 1)
    rrow = lax.broadcasted_iota(jnp.int32, (rows, 128), 0)
    picked = jnp.where(rlane == (rrow & 127), mrep, 0.0)
    mcol = jnp.sum(picked, axis=1, keepdims=True)
    o_ref[...] = jnp.where(mcol > 0.5, shifted, xb)


def _flip_rows(x, mask):
    rows = 4096
    grid = N // rows
    return pl.pallas_call(
        _flip_body,
        grid=(grid,),
        in_specs=[
            pl.BlockSpec((rows, D), lambda i: (i, 0)),
            pl.BlockSpec((rows // 128, 128), lambda i: (i, 0)),
        ],
        out_specs=pl.BlockSpec((rows, D), lambda i: (i, 0)),
        out_shape=jax.ShapeDtypeStruct((N, D), jnp.float32),
    )(x, mask)


@jax.jit
def kernel(x, indices):
    mask = _build_mask(indices)
    return _flip_rows(x, mask.reshape(N // 128, 128))


# speedup vs baseline: 1.0314x; 1.0314x over previous
import functools

import jax
import jax.numpy as jnp
from jax import lax
from jax.experimental import pallas as pl
from jax.experimental.pallas import tpu as pltpu
from jax.experimental.pallas import tpu_sc as plsc

N = 65536
D = 256
OFF = 6

_NC = 2
_NS = 16
_NW = _NC * _NS
_L = 16
_SLAB = N // _NW


def _mask_body(idx_hbm, mask_hbm, idx_v, slab_v):
    wid = lax.axis_index("s") * _NC + lax.axis_index("c")
    lo = wid * _SLAB

    pltpu.sync_copy(idx_hbm, idx_v)

    unroll = 8
    zeros = jnp.zeros((_L,), jnp.float32)

    def zero_body(i, carry):
        for u in range(unroll):
            slab_v[pl.ds((i * unroll + u) * _L, _L)] = zeros
        return carry

    lax.fori_loop(0, _SLAB // (_L * unroll), zero_body, 0)

    ones = jnp.ones((_L,), jnp.float32)
    n_idx = idx_v.shape[0]

    def scatter_body(i, carry):
        for u in range(unroll):
            v = idx_v[pl.ds((i * unroll + u) * _L, _L)]
            rel = v - lo
            m = plsc.bitcast(rel, jnp.uint32) < _SLAB
            plsc.store_scatter(slab_v, [rel & (_SLAB - 1)], ones, mask=m)
        return carry

    lax.fori_loop(0, n_idx // (_L * unroll), scatter_body, 0)

    pltpu.sync_copy(slab_v, mask_hbm.at[pl.ds(lo, _SLAB)])


def _build_mask(indices):
    n_idx = indices.shape[0]
    mesh = plsc.VectorSubcoreMesh(core_axis_name="c", subcore_axis_name="s")
    kern = pl.kernel(
        _mask_body,
        out_type=jax.ShapeDtypeStruct((N,), jnp.float32),
        mesh=mesh,
        scratch_types=[
            pltpu.VMEM((n_idx,), jnp.int32),
            pltpu.VMEM((_SLAB,), jnp.float32),
        ],
        compiler_params=pltpu.CompilerParams(needs_layout_passes=False),
    )
    return kern(indices)


def _flip_body(x_ref, m_ref, o_ref):
    H = D // 2
    xb = x_ref[...]
    a = xb[:, :, :H]
    b = xb[:, :, H:]
    k = lax.broadcasted_iota(jnp.int32, a.shape, 2)
    idxg = jnp.where(k < OFF, OFF - 1 - k, H + OFF - 1 - k)
    ga = jnp.take_along_axis(a, idxg, axis=2)
    gb = jnp.take_along_axis(b, idxg, axis=2)
    out_a = jnp.where(k < OFF, a, gb)
    out_b = jnp.where(k < OFF, gb, ga)
    shifted = jnp.concatenate([out_a, out_b], axis=2)
    mcol = m_ref[...][:, :, None]
    o_ref[...] = jnp.where(mcol > 0.5, shifted, xb)


def _flip_rows(x, mask):
    rows = 8192
    g = rows // 128
    grid = N // rows
    out3 = pl.pallas_call(
        _flip_body,
        grid=(grid,),
        in_specs=[
            pl.BlockSpec((g, 128, D), lambda i: (i, 0, 0)),
            pl.BlockSpec((g, 128), lambda i: (i, 0)),
        ],
        out_specs=pl.BlockSpec((g, 128, D), lambda i: (i, 0, 0)),
        out_shape=jax.ShapeDtypeStruct((N // 128, 128, D), jnp.float32),
    )(x.reshape(N // 128, 128, D), mask)
    return out3.reshape(N, D)


@jax.jit
def kernel(x, indices):
    mask = _build_mask(indices)
    return _flip_rows(x, mask.reshape(N // 128, 128))
